# baseline (speedup 1.0000x reference)
import functools
import math

import jax
import jax.numpy as jnp
from jax import lax
from jax.experimental import pallas as pl
from jax.experimental.pallas import tpu as pltpu
from jax.experimental.pallas import tpu_sc as plsc

D_MODEL = 64
PADDED = 128
SCALE = math.sqrt(D_MODEL)

NUM_CORES = 2
NUM_SUBCORES = 16
NUM_WORKERS = NUM_CORES * NUM_SUBCORES

CHUNK = 128
GATHER_ROWS = 128
SUBGATHERS = CHUNK // GATHER_ROWS
LANES = 16


VOCAB = 1000000
TBLOCK = 1024


def _pad_block_kernel(lutT_ref, out_ref):
    t = jnp.transpose(lutT_ref[...], (1, 0)) * SCALE
    out_ref[...] = jnp.concatenate(
        [t, jnp.zeros((TBLOCK, PADDED - D_MODEL), jnp.float32)], axis=1)


def _build_pad_kernel():
    n_blocks = (VOCAB + TBLOCK - 1) // TBLOCK
    return pl.pallas_call(
        _pad_block_kernel,
        grid=(n_blocks,),
        in_specs=[pl.BlockSpec((D_MODEL, TBLOCK), lambda i: (0, i))],
        out_specs=pl.BlockSpec((TBLOCK, PADDED), lambda i: (i, 0)),
        out_shape=jax.ShapeDtypeStruct((VOCAB, PADDED), jnp.float32),
    )


def _build_kernel(n_rows):
    assert n_rows % (NUM_WORKERS * CHUNK) == 0
    rows_per_worker = n_rows // NUM_WORKERS
    n_chunks = rows_per_worker // CHUNK
    assert n_chunks % 2 == 0
    mesh = plsc.VectorSubcoreMesh(core_axis_name="c", subcore_axis_name="s")

    @functools.partial(
        pl.kernel,
        mesh=mesh,
        compiler_params=pltpu.CompilerParams(use_tc_tiling_on_sc=True),
        out_type=jax.ShapeDtypeStruct((n_rows, D_MODEL), jnp.float32),
        scratch_types=[
            pltpu.VMEM((rows_per_worker,), jnp.int32),
            pltpu.VMEM((CHUNK, PADDED), jnp.float32),
            pltpu.VMEM((CHUNK, PADDED), jnp.float32),
            pltpu.VMEM((CHUNK, D_MODEL), jnp.float32),
            pltpu.VMEM((CHUNK, D_MODEL), jnp.float32),
            pltpu.SemaphoreType.DMA,
            pltpu.SemaphoreType.DMA,
            pltpu.SemaphoreType.DMA,
            pltpu.SemaphoreType.DMA,
        ],
    )
    def emb_kernel(idx_hbm, lutp_hbm, out_hbm,
                   idx_v, buf0, buf1, obuf0, obuf1,
                   gsem0, gsem1, wsem0, wsem1):
        wid = lax.axis_index("s") * NUM_CORES + lax.axis_index("c")
        base = wid * rows_per_worker
        pltpu.sync_copy(idx_hbm.at[pl.ds(base, rows_per_worker)], idx_v)

        def start_gather(chunk, buf, gsem):
            for j in range(SUBGATHERS):
                idx_slice = idx_v.at[pl.ds(chunk * CHUNK + j * GATHER_ROWS,
                                           GATHER_ROWS)]
                pltpu.async_copy(
                    lutp_hbm.at[idx_slice],
                    buf.at[pl.ds(j * GATHER_ROWS, GATHER_ROWS)],
                    gsem,
                )

        def wait_gather(buf, gsem):
            pltpu.make_async_copy(
                lutp_hbm.at[pl.ds(0, CHUNK)], buf, gsem).wait()

        def scale_compact(buf, obuf):
            def row_body(i, carry):
                for j in range(D_MODEL // LANES):
                    sl = pl.ds(j * LANES, LANES)
                    obuf[i, sl] = buf[i, sl]
                return carry

            lax.fori_loop(0, CHUNK, row_body, 0)

        def start_write(chunk, obuf, wsem):
            pltpu.async_copy(
                obuf, out_hbm.at[pl.ds(base + chunk * CHUNK, CHUNK)], wsem)

        def wait_write(obuf, wsem):
            pltpu.make_async_copy(
                obuf, out_hbm.at[pl.ds(0, CHUNK)], wsem).wait()

        def finish_chunk(chunk, buf, obuf, gsem, wsem):
            wait_gather(buf, gsem)
            scale_compact(buf, obuf)
            start_write(chunk, obuf, wsem)

        start_gather(0, buf0, gsem0)

        def pair_body(p, carry):
            g = p * 2

            @pl.when(p > 0)
            def _():
                wait_write(obuf1, wsem1)

            start_gather(g + 1, buf1, gsem1)

            @pl.when(p > 0)
            def _():
                wait_write(obuf0, wsem0)

            finish_chunk(g, buf0, obuf0, gsem0, wsem0)

            @pl.when(g + 2 < n_chunks)
            def _():
                start_gather(g + 2, buf0, gsem0)

            finish_chunk(g + 1, buf1, obuf1, gsem1, wsem1)
            return carry

        lax.fori_loop(0, n_chunks // 2, pair_body, 0)
        wait_write(obuf0, wsem0)
        wait_write(obuf1, wsem1)

    return emb_kernel


def kernel(x, lut):
    n_rows = x.shape[0] * x.shape[1]
    flat_idx = x.reshape(n_rows)
    lutp = _build_pad_kernel()(lut.T)
    out = _build_kernel(n_rows)(flat_idx, lutp)
    return out.reshape(x.shape[0], x.shape[1], D_MODEL)


# speedup vs baseline: 1.1858x; 1.1858x over previous
import functools
import math

import jax
import jax.numpy as jnp
from jax import lax
from jax.experimental import pallas as pl
from jax.experimental.pallas import tpu as pltpu
from jax.experimental.pallas import tpu_sc as plsc

D_MODEL = 64
PADDED = 128
SCALE = math.sqrt(D_MODEL)

NUM_CORES = 2
NUM_SUBCORES = 16
NUM_WORKERS = NUM_CORES * NUM_SUBCORES

CHUNK = 128
GATHER_ROWS = 128
SUBGATHERS = CHUNK // GATHER_ROWS
LANES = 16


def _pad(lut):
    return jnp.pad(lut, ((0, 0), (0, PADDED - D_MODEL)))


def _build_kernel(n_rows):
    assert n_rows % (NUM_WORKERS * CHUNK) == 0
    rows_per_worker = n_rows // NUM_WORKERS
    n_chunks = rows_per_worker // CHUNK
    assert n_chunks % 2 == 0
    mesh = plsc.VectorSubcoreMesh(core_axis_name="c", subcore_axis_name="s")

    @functools.partial(
        pl.kernel,
        mesh=mesh,
        compiler_params=pltpu.CompilerParams(use_tc_tiling_on_sc=True),
        out_type=jax.ShapeDtypeStruct((n_rows, D_MODEL), jnp.float32),
        scratch_types=[
            pltpu.VMEM((rows_per_worker,), jnp.int32),
            pltpu.VMEM((CHUNK, PADDED), jnp.float32),
            pltpu.VMEM((CHUNK, PADDED), jnp.float32),
            pltpu.VMEM((CHUNK, D_MODEL), jnp.float32),
            pltpu.VMEM((CHUNK, D_MODEL), jnp.float32),
            pltpu.SemaphoreType.DMA,
            pltpu.SemaphoreType.DMA,
            pltpu.SemaphoreType.DMA,
            pltpu.SemaphoreType.DMA,
        ],
    )
    def emb_kernel(idx_hbm, lutp_hbm, out_hbm,
                   idx_v, buf0, buf1, obuf0, obuf1,
                   gsem0, gsem1, wsem0, wsem1):
        wid = lax.axis_index("s") * NUM_CORES + lax.axis_index("c")
        base = wid * rows_per_worker
        pltpu.sync_copy(idx_hbm.at[pl.ds(base, rows_per_worker)], idx_v)

        def start_gather(chunk, buf, gsem):
            for j in range(SUBGATHERS):
                idx_slice = idx_v.at[pl.ds(chunk * CHUNK + j * GATHER_ROWS,
                                           GATHER_ROWS)]
                pltpu.async_copy(
                    lutp_hbm.at[idx_slice],
                    buf.at[pl.ds(j * GATHER_ROWS, GATHER_ROWS)],
                    gsem,
                )

        def wait_gather(buf, gsem):
            pltpu.make_async_copy(
                lutp_hbm.at[pl.ds(0, CHUNK)], buf, gsem).wait()

        def scale_compact(buf, obuf):
            def row_body(i, carry):
                for j in range(D_MODEL // LANES):
                    sl = pl.ds(j * LANES, LANES)
                    obuf[i, sl] = buf[i, sl] * SCALE
                return carry

            lax.fori_loop(0, CHUNK, row_body, 0)

        def start_write(chunk, obuf, wsem):
            pltpu.async_copy(
                obuf, out_hbm.at[pl.ds(base + chunk * CHUNK, CHUNK)], wsem)

        def wait_write(obuf, wsem):
            pltpu.make_async_copy(
                obuf, out_hbm.at[pl.ds(0, CHUNK)], wsem).wait()

        def finish_chunk(chunk, buf, obuf, gsem, wsem):
            wait_gather(buf, gsem)
            scale_compact(buf, obuf)
            start_write(chunk, obuf, wsem)

        start_gather(0, buf0, gsem0)

        def pair_body(p, carry):
            g = p * 2

            @pl.when(p > 0)
            def _():
                wait_write(obuf1, wsem1)

            start_gather(g + 1, buf1, gsem1)

            @pl.when(p > 0)
            def _():
                wait_write(obuf0, wsem0)

            finish_chunk(g, buf0, obuf0, gsem0, wsem0)

            @pl.when(g + 2 < n_chunks)
            def _():
                start_gather(g + 2, buf0, gsem0)

            finish_chunk(g + 1, buf1, obuf1, gsem1, wsem1)
            return carry

        lax.fori_loop(0, n_chunks // 2, pair_body, 0)
        wait_write(obuf0, wsem0)
        wait_write(obuf1, wsem1)

    return emb_kernel


def kernel(x, lut):
    n_rows = x.shape[0] * x.shape[1]
    flat_idx = x.reshape(n_rows)
    lutp = jax.jit(_pad)(lut)
    out = _build_kernel(n_rows)(flat_idx, lutp)
    return out.reshape(x.shape[0], x.shape[1], D_MODEL)
